# baseline (speedup 1.0000x reference)
import functools

import jax
import jax.numpy as jnp
from jax import lax
from jax.experimental import pallas as pl
from jax.experimental.pallas import tpu as pltpu
from jax.experimental.pallas import tpu_sc as plsc

_NUM_EMBEDDINGS = 1024
_EMBEDDING_DIM = 64
_COMMITMENT_COST = 0.25
_LB = 4096
_NC = 2
_NS = 16
_NW = _NC * _NS


def _vq_kernel(x_ref, w2_ref, wsq_ref, wt_ref, out_ref, loss_ref, idx_ref,
               sse_ref, *, n_elems):
    b = pl.program_id(0)
    l = pl.program_id(1)
    nb = pl.num_programs(0)
    nl = pl.num_programs(1)

    @pl.when((b == 0) & (l == 0))
    def _init():
        sse_ref[0] = 0.0

    x = x_ref[...]
    d = jnp.dot(w2_ref[...], x, preferred_element_type=jnp.float32) + wsq_ref[...]
    idx = jnp.argmin(d, axis=0)[None, :]
    idx_ref[...] = idx.reshape(1, 1, -1)
    iota = jax.lax.broadcasted_iota(jnp.int32, d.shape, 0)
    onehot = (iota == idx).astype(jnp.float32)
    q = jnp.dot(wt_ref[...], onehot, preferred_element_type=jnp.float32)
    out_ref[...] = q
    sse_ref[0] += jnp.sum((q - x) * (q - x))

    @pl.when((b == nb - 1) & (l == nl - 1))
    def _finalize():
        loss = (1.0 + _COMMITMENT_COST) * sse_ref[0] / n_elems
        loss_ref[...] = jnp.reshape(loss, (1, 1))


def _hist_sc(n_idx):
    per_w = n_idx // _NW
    mesh = plsc.VectorSubcoreMesh(core_axis_name="c", subcore_axis_name="s")

    @functools.partial(
        pl.kernel, mesh=mesh,
        out_type=jax.ShapeDtypeStruct((_NC, _NUM_EMBEDDINGS), jnp.float32),
        scratch_types=[
            pltpu.VMEM((per_w,), jnp.int32),
            pltpu.VMEM((per_w,), jnp.float32),
            pltpu.VMEM_SHARED((_NUM_EMBEDDINGS,), jnp.float32),
        ],
    )
    def hist(idx_hbm, zeros_hbm, ones_hbm, out_hbm, idx_v, ones_v, shared):
        c = lax.axis_index("c")
        s = lax.axis_index("s")
        wid = s * _NC + c

        @pl.when(s == 0)
        def _zero():
            pltpu.sync_copy(zeros_hbm, shared)

        pltpu.sync_copy(ones_hbm, ones_v)
        pltpu.sync_copy(idx_hbm.at[pl.ds(wid * per_w, per_w)], idx_v)
        plsc.subcore_barrier()
        pltpu.sync_copy(ones_v, shared.at[idx_v], add=True)
        plsc.subcore_barrier()

        @pl.when(s == 0)
        def _drain():
            pltpu.sync_copy(shared, out_hbm.at[c])

    return hist


def _perp_kernel(h_ref, perp_ref, *, n_rows):
    counts = jnp.sum(h_ref[...], axis=0, keepdims=True)
    p = counts / n_rows
    ent = jnp.sum(p * jnp.log(p + 1e-10))
    perp_ref[...] = jnp.reshape(jnp.exp(-ent), (1, 1))


def kernel(inputs, weight):
    batch, c, length = inputs.shape
    n_rows = batch * length
    n_elems = batch * length * c

    last = jnp.transpose(inputs[-1, :, length - _NUM_EMBEDDINGS:], (1, 0))
    w = jnp.where(jnp.all(weight == 0.0), last, weight)

    x2d = inputs.reshape(batch * c, length)
    grid = (batch, length // _LB)
    nl = length // _LB
    body = functools.partial(_vq_kernel, n_elems=float(n_elems))
    q, loss, idx3 = pl.pallas_call(
        body,
        grid=grid,
        in_specs=[
            pl.BlockSpec((c, _LB), lambda b, l: (b, l)),
            pl.BlockSpec((_NUM_EMBEDDINGS, _EMBEDDING_DIM), lambda b, l: (0, 0)),
            pl.BlockSpec((_NUM_EMBEDDINGS, 1), lambda b, l: (0, 0)),
            pl.BlockSpec((_EMBEDDING_DIM, _NUM_EMBEDDINGS), lambda b, l: (0, 0)),
        ],
        out_specs=[
            pl.BlockSpec((c, _LB), lambda b, l: (b, l)),
            pl.BlockSpec((1, 1), lambda b, l: (0, 0)),
            pl.BlockSpec((1, 1, _LB), lambda b, l: (b * nl + l, 0, 0)),
        ],
        out_shape=[
            jax.ShapeDtypeStruct((batch * c, length), jnp.float32),
            jax.ShapeDtypeStruct((1, 1), jnp.float32),
            jax.ShapeDtypeStruct((batch * nl, 1, _LB), jnp.int32),
        ],
        scratch_shapes=[
            pltpu.SMEM((1,), jnp.float32),
        ],
        compiler_params=pltpu.CompilerParams(
            dimension_semantics=("arbitrary", "arbitrary")),
    )(x2d, -2.0 * w, jnp.sum(w * w, axis=1, keepdims=True), w.T)

    hist = _hist_sc(n_rows)(
        idx3.reshape(-1),
        jnp.zeros((_NUM_EMBEDDINGS,), jnp.float32),
        jnp.ones((n_rows // _NW,), jnp.float32),
    )

    perp = pl.pallas_call(
        functools.partial(_perp_kernel, n_rows=float(n_rows)),
        out_shape=jax.ShapeDtypeStruct((1, 1), jnp.float32),
    )(hist)

    return (loss[0, 0], q.reshape(batch, c, length), perp[0, 0])
